# baseline (speedup 1.0000x reference)
import functools

import jax
import jax.numpy as jnp
from jax import lax
from jax.experimental import pallas as pl
from jax.experimental.pallas import tpu as pltpu
from jax.experimental.pallas import tpu_sc as plsc

_N = 1000000
_B = 16384
_DIH_EPS = 0.076
_GNN_EPS = 6.38
_FS_EPS = 3.0
_LR = 0.001

_NS = 16
_PB = _B // _NS
_RB = _PB // 128
_SUB = 15616
_NSUB = 4
_COPY = _SUB * _NSUB
_TAIL = _N - _NS * _COPY


def _sc_body(idx_hbm, dih_hbm, gnn_hbm, fs_hbm, lamd_hbm, lamg_hbm, lamf_hbm,
             outd_hbm, outg_hbm, outf_hbm, part_hbm,
             half0, half1, tailb, idx_v, loss_a, loss_b, lam_a, lam_b,
             new_a, new_b, pacc,
             semi0, semi1, semo0, semo1, sem_g):
  cid = lax.axis_index("c")
  sid = lax.axis_index("s")
  zero = jnp.zeros((16,), jnp.float32)
  halves = (half0, half1)
  semi = (semi0, semi1)
  semo = (semo0, semo1)

  def per_core(bufs, zero_rows):
    pltpu.sync_copy(idx_hbm.at[sid], idx_v)
    gath = [pltpu.async_copy(lam_hbm.at[idx_v], lam_v, sem_g)
            for (_, lam_hbm, _, _, _, lam_v, _, _) in bufs]

    subs = []
    for (_, lam_hbm, out_hbm, _, _, _, _, _) in bufs:
      for i in range(_NSUB):
        off = sid * _COPY + i * _SUB
        subs.append((lam_hbm, out_hbm, off))
    outs = [None, None]
    for i, (src, dst, off) in enumerate(subs):
      h = i % 2
      if outs[h] is not None:
        outs[h].wait()
      pltpu.async_copy(src.at[pl.ds(off, _SUB)], halves[h], semi[h]).wait()
      outs[h] = pltpu.async_copy(halves[h], dst.at[pl.ds(off, _SUB)], semo[h])
    for cp in outs:
      if cp is not None:
        cp.wait()

    @pl.when(sid == 0)
    def _():
      for (_, lam_hbm, out_hbm, _, _, _, _, _) in bufs:
        pltpu.sync_copy(lam_hbm.at[pl.ds(_NS * _COPY, _TAIL)], tailb)
        pltpu.sync_copy(tailb, out_hbm.at[pl.ds(_NS * _COPY, _TAIL)])

    for cp in gath:
      cp.wait()
    for (loss_hbm, _, _, eps, loss_v, lam_v, new_v, row) in bufs:
      pltpu.sync_copy(loss_hbm.at[sid], loss_v)
      acc = jnp.zeros((16,), jnp.float32)
      for k in range(_PB // 16):
        lam = lam_v[pl.ds(k * 16, 16)]
        viol = loss_v[pl.ds(k * 16, 16)] - eps
        acc = acc + lam * viol
        new_v[pl.ds(k * 16, 16)] = jnp.maximum(lam + _LR * viol, 0.0)
      pacc[row, pl.ds(0, 16)] = acc
    for row in zero_rows:
      pacc[row, pl.ds(0, 16)] = zero

    plsc.subcore_barrier()

    scs = [pltpu.async_copy(new_v, out_hbm.at[idx_v], sem_g)
           for (_, _, out_hbm, _, _, _, new_v, _) in bufs]
    for cp in scs:
      cp.wait()

    pltpu.sync_copy(pacc, part_hbm.at[cid, sid])

  @pl.when(cid == 0)
  def _():
    per_core([(dih_hbm, lamd_hbm, outd_hbm, _DIH_EPS, loss_a, lam_a, new_a, 0)],
             zero_rows=(1, 2))

  @pl.when(cid == 1)
  def _():
    per_core([(gnn_hbm, lamg_hbm, outg_hbm, _GNN_EPS, loss_a, lam_a, new_a, 1),
              (fs_hbm, lamf_hbm, outf_hbm, _FS_EPS, loss_b, lam_b, new_b, 2)],
             zero_rows=(0,))


_sc_call = functools.partial(
    pl.kernel,
    out_type=(
        jax.ShapeDtypeStruct((_N,), jnp.float32),
        jax.ShapeDtypeStruct((_N,), jnp.float32),
        jax.ShapeDtypeStruct((_N,), jnp.float32),
        jax.ShapeDtypeStruct((2, _NS, 3, 16), jnp.float32),
    ),
    mesh=plsc.VectorSubcoreMesh(core_axis_name="c", subcore_axis_name="s"),
    scratch_types=[
        pltpu.VMEM((_SUB,), jnp.float32),
        pltpu.VMEM((_SUB,), jnp.float32),
        pltpu.VMEM((_TAIL,), jnp.float32),
        pltpu.VMEM((_PB,), jnp.int32),
        pltpu.VMEM((_PB,), jnp.float32),
        pltpu.VMEM((_PB,), jnp.float32),
        pltpu.VMEM((_PB,), jnp.float32),
        pltpu.VMEM((_PB,), jnp.float32),
        pltpu.VMEM((_PB,), jnp.float32),
        pltpu.VMEM((_PB,), jnp.float32),
        pltpu.VMEM((3, 16), jnp.float32),
        pltpu.SemaphoreType.DMA,
        pltpu.SemaphoreType.DMA,
        pltpu.SemaphoreType.DMA,
        pltpu.SemaphoreType.DMA,
        pltpu.SemaphoreType.DMA,
    ],
)(_sc_body)


def kernel(primary_loss, dihedral_losses, gnn_losses, foldseek_losses, indices,
           lam_dihedral, lam_gnn, lam_foldseek):
  idx3 = indices.astype(jnp.int32).reshape(_NS, _PB)
  dih3 = dihedral_losses.reshape(_NS, _PB)
  gnn3 = gnn_losses.reshape(_NS, _PB)
  fs3 = foldseek_losses.reshape(_NS, _PB)
  out_d, out_g, out_f, part = _sc_call(
      idx3, dih3, gnn3, fs3, lam_dihedral, lam_gnn, lam_foldseek)
  lagrangian = primary_loss + jnp.sum(part) / jnp.float32(_B)
  return lagrangian, out_d, out_g, out_f


# speedup vs baseline: 1.0425x; 1.0425x over previous
import functools

import jax
import jax.numpy as jnp
from jax import lax
from jax.experimental import pallas as pl
from jax.experimental.pallas import tpu as pltpu
from jax.experimental.pallas import tpu_sc as plsc

_N = 1000000
_B = 16384
_DIH_EPS = 0.076
_GNN_EPS = 6.38
_FS_EPS = 3.0
_LR = 0.001

_NS = 16
_PB = _B // _NS
_SUB = 15616
_NSUB = 4
_COPY = _SUB * _NSUB
_TAIL = _N - _NS * _COPY


def _sc_body(idx_hbm, dih_hbm, gnn_hbm, fs_hbm, lamd_hbm, lamg_hbm, lamf_hbm,
             outd_hbm, outg_hbm, outf_hbm, part_hbm,
             half0, half1, tailb, idx_v, loss_a, loss_b, lam_a, lam_b,
             new_a, new_b, pacc,
             semi0, semi1, semo0, semo1, sem_g):
  cid = lax.axis_index("c")
  sid = lax.axis_index("s")
  zero = jnp.zeros((16,), jnp.float32)
  halves = (half0, half1)
  semi = (semi0, semi1)
  semo = (semo0, semo1)

  def per_core(bufs, zero_rows):
    pltpu.sync_copy(idx_hbm.at[pl.ds(sid * _PB, _PB)], idx_v)
    gath = [pltpu.async_copy(lam_hbm.at[idx_v], lam_v, sem_g)
            for (_, lam_hbm, _, _, _, lam_v, _, _) in bufs]

    subs = []
    for (_, lam_hbm, out_hbm, _, _, _, _, _) in bufs:
      for i in range(_NSUB):
        off = sid * _COPY + i * _SUB
        subs.append((lam_hbm, out_hbm, off))
    outs = [None, None]
    for i, (src, dst, off) in enumerate(subs):
      h = i % 2
      if outs[h] is not None:
        outs[h].wait()
      pltpu.async_copy(src.at[pl.ds(off, _SUB)], halves[h], semi[h]).wait()
      outs[h] = pltpu.async_copy(halves[h], dst.at[pl.ds(off, _SUB)], semo[h])
    for cp in outs:
      if cp is not None:
        cp.wait()

    @pl.when(sid == 0)
    def _():
      for (_, lam_hbm, out_hbm, _, _, _, _, _) in bufs:
        pltpu.sync_copy(lam_hbm.at[pl.ds(_NS * _COPY, _TAIL)], tailb)
        pltpu.sync_copy(tailb, out_hbm.at[pl.ds(_NS * _COPY, _TAIL)])

    for cp in gath:
      cp.wait()
    for (loss_hbm, _, _, eps, loss_v, lam_v, new_v, row) in bufs:
      pltpu.sync_copy(loss_hbm.at[pl.ds(sid * _PB, _PB)], loss_v)

      def step(k, acc):
        o = pl.multiple_of(k * 16, 16)
        lam = lam_v[pl.ds(o, 16)]
        viol = loss_v[pl.ds(o, 16)] - eps
        new_v[pl.ds(o, 16)] = jnp.maximum(lam + _LR * viol, 0.0)
        return acc + lam * viol

      acc = lax.fori_loop(0, _PB // 16, step, jnp.zeros((16,), jnp.float32))
      pacc[row, pl.ds(0, 16)] = acc
    for row in zero_rows:
      pacc[row, pl.ds(0, 16)] = zero

    plsc.subcore_barrier()

    scs = [pltpu.async_copy(new_v, out_hbm.at[idx_v], sem_g)
           for (_, _, out_hbm, _, _, _, new_v, _) in bufs]
    for cp in scs:
      cp.wait()

    pltpu.sync_copy(pacc, part_hbm.at[cid, sid])

  @pl.when(cid == 0)
  def _():
    per_core([(dih_hbm, lamd_hbm, outd_hbm, _DIH_EPS, loss_a, lam_a, new_a, 0)],
             zero_rows=(1, 2))

  @pl.when(cid == 1)
  def _():
    per_core([(gnn_hbm, lamg_hbm, outg_hbm, _GNN_EPS, loss_a, lam_a, new_a, 1),
              (fs_hbm, lamf_hbm, outf_hbm, _FS_EPS, loss_b, lam_b, new_b, 2)],
             zero_rows=(0,))


_sc_call = functools.partial(
    pl.kernel,
    out_type=(
        jax.ShapeDtypeStruct((_N,), jnp.float32),
        jax.ShapeDtypeStruct((_N,), jnp.float32),
        jax.ShapeDtypeStruct((_N,), jnp.float32),
        jax.ShapeDtypeStruct((2, _NS, 3, 16), jnp.float32),
    ),
    mesh=plsc.VectorSubcoreMesh(core_axis_name="c", subcore_axis_name="s"),
    scratch_types=[
        pltpu.VMEM((_SUB,), jnp.float32),
        pltpu.VMEM((_SUB,), jnp.float32),
        pltpu.VMEM((_TAIL,), jnp.float32),
        pltpu.VMEM((_PB,), jnp.int32),
        pltpu.VMEM((_PB,), jnp.float32),
        pltpu.VMEM((_PB,), jnp.float32),
        pltpu.VMEM((_PB,), jnp.float32),
        pltpu.VMEM((_PB,), jnp.float32),
        pltpu.VMEM((_PB,), jnp.float32),
        pltpu.VMEM((_PB,), jnp.float32),
        pltpu.VMEM((3, 16), jnp.float32),
        pltpu.SemaphoreType.DMA,
        pltpu.SemaphoreType.DMA,
        pltpu.SemaphoreType.DMA,
        pltpu.SemaphoreType.DMA,
        pltpu.SemaphoreType.DMA,
    ],
)(_sc_body)


def kernel(primary_loss, dihedral_losses, gnn_losses, foldseek_losses, indices,
           lam_dihedral, lam_gnn, lam_foldseek):
  out_d, out_g, out_f, part = _sc_call(
      indices.astype(jnp.int32), dihedral_losses, gnn_losses, foldseek_losses,
      lam_dihedral, lam_gnn, lam_foldseek)
  lagrangian = primary_loss + jnp.sum(part) / jnp.float32(_B)
  return lagrangian, out_d, out_g, out_f
